# baseline (speedup 1.0000x reference)
import functools

import jax
import jax.numpy as jnp
from jax import lax
from jax.experimental import pallas as pl
from jax.experimental.pallas import tpu as pltpu
from jax.experimental.pallas import tpu_sc as plsc

NUM_TYPES = 11
B = 1024
M = 4096
Q = M // 4
LANES = 16
NW = 32
SPW = B // NW
NVR = M // LANES
RB = 128

_mesh = plsc.VectorSubcoreMesh(core_axis_name="c", subcore_axis_name="s")


def _pack_body(res_ref, atm_ref, out_ref):
    s = res_ref[...] + atm_ref[...]
    s = s - jnp.where(s >= 44, 44, 0)
    s = s - jnp.where(s >= 22, 22, 0)
    s = s - jnp.where(s >= 11, 11, 0)
    out_ref[...] = (
        s[:, 0:Q]
        | (s[:, Q:2 * Q] << 8)
        | (s[:, 2 * Q:3 * Q] << 16)
        | (s[:, 3 * Q:4 * Q] << 24)
    )


_pack_types_tc = pl.pallas_call(
    _pack_body,
    grid=(B // RB,),
    in_specs=[
        pl.BlockSpec((RB, M), lambda i: (i, 0)),
        pl.BlockSpec((RB, M), lambda i: (i, 0)),
    ],
    out_specs=pl.BlockSpec((RB, Q), lambda i: (i, 0)),
    out_shape=jax.ShapeDtypeStruct((B, Q), jnp.int32),
)


@functools.partial(
    pl.kernel,
    out_type=[
        jax.ShapeDtypeStruct((B // 2, 6 * M), jnp.float32),
        jax.ShapeDtypeStruct((B * NUM_TYPES,), jnp.int32),
        jax.ShapeDtypeStruct((B * NUM_TYPES,), jnp.int32),
    ],
    mesh=_mesh,
    scratch_types=[
        pltpu.VMEM((2 * Q,), jnp.int32),
        pltpu.VMEM((2 * Q,), jnp.int32),
        pltpu.VMEM((6 * M,), jnp.float32),
        pltpu.VMEM((6 * M,), jnp.float32),
        pltpu.VMEM((6 * M,), jnp.float32),
        pltpu.VMEM((6 * M,), jnp.float32),
        pltpu.VMEM((M,), jnp.int32),
        pltpu.VMEM((M,), jnp.int32),
        pltpu.VMEM((LANES,), jnp.int32),
        pltpu.VMEM((SPW,), jnp.int32),
        pltpu.VMEM((SPW * NUM_TYPES,), jnp.int32),
        pltpu.VMEM((SPW * NUM_TYPES,), jnp.int32),
        pltpu.SemaphoreType.DMA,
        pltpu.SemaphoreType.DMA,
        pltpu.SemaphoreType.DMA,
        pltpu.SemaphoreType.DMA,
    ],
    compiler_params=pltpu.CompilerParams(needs_layout_passes=False),
)
def _typed_coords_sc(
    crd_hbm, typ_hbm, na_hbm,
    outc_hbm, cnts_hbm, offs_hbm,
    typ_v0, typ_v1, crd_v0, crd_v1, out_v0, out_v1,
    enc_v, hst_v, off_v, na_v, co_v, of_v,
    sin0, sin1, sout0, sout1,
):
    typ_b = (typ_v0, typ_v1)
    crd_b = (crd_v0, crd_v1)
    out_b = (out_v0, out_v1)
    sin = (sin0, sin1)
    sout = (sout0, sout1)

    wid = lax.axis_index("c") * 16 + lax.axis_index("s")
    base = wid * SPW
    rbase = wid * (SPW // 2)
    pltpu.sync_copy(na_hbm.at[pl.ds(base, SPW)], na_v)
    iota = lax.broadcasted_iota(jnp.int32, (LANES,), 0)
    zero_f = jnp.zeros((LANES,), jnp.float32)

    def issue_in(ri, k):
        row = rbase + ri
        pltpu.async_copy(typ_hbm.at[row], typ_b[k], sin[k])
        pltpu.async_copy(crd_hbm.at[row], crd_b[k], sin[k])

    def wait_in(ri, k):
        row = rbase + ri
        pltpu.make_async_copy(typ_hbm.at[row], typ_b[k], sin[k]).wait()
        pltpu.make_async_copy(crd_hbm.at[row], crd_b[k], sin[k]).wait()

    def compute_sample(ri, k, u):
        typ_v, crd_v, out_v = typ_b[k], crd_b[k], out_b[k]
        si = ri * 2 + u
        na_sp = plsc.load_gather(na_v, [jnp.zeros((LANES,), jnp.int32) + si])

        @plsc.parallel_loop(0, NVR, unroll=8)
        def zero_hist(j):
            hst_v[pl.ds(j * LANES, LANES)] = jnp.zeros((LANES,), jnp.int32)

        @plsc.parallel_loop(0, NVR // 4)
        def pass_a(i):
            w = typ_v[pl.ds(u * Q + i * LANES, LANES)]
            for q in range(4):
                j = q * (NVR // 4) + i
                t = (w >> (8 * q)) & 0xFF
                gi = j * LANES + iota
                t = jnp.where(gi < na_sp, t, NUM_TYPES)
                c1, lastm = plsc.scan_count(t)
                enc_v[pl.ds(j * LANES, LANES)] = t * LANES + (c1 - 1)
                plsc.store_scatter(hst_v, [j * LANES + t], c1, mask=lastm)

        def prefix(j, acc):
            sl = pl.ds(j * LANES, LANES)
            hv = hst_v[sl]
            hst_v[sl] = acc
            return acc + hv

        c = lax.fori_loop(0, NVR, prefix, jnp.zeros((LANES,), jnp.int32),
                          unroll=4)
        off = plsc.cumsum(c) - c
        off_v[...] = off
        dsti = si * NUM_TYPES + iota
        m11 = iota < NUM_TYPES
        plsc.store_scatter(co_v, [dsti], c, mask=m11)
        plsc.store_scatter(of_v, [dsti], off, mask=m11)

        @plsc.parallel_loop(0, NVR, unroll=2)
        def pass_b(j):
            sl = pl.ds(j * LANES, LANES)
            enc = enc_v[sl]
            t = enc >> 4
            pos = (plsc.load_gather(off_v, [t])
                   + plsc.load_gather(hst_v, [j * LANES + t])
                   + (enc & (LANES - 1)))
            cb = u * 3 * M
            src3 = cb + (j * LANES) * 3 + iota * 3
            x = plsc.load_gather(crd_v, [src3])
            y = plsc.load_gather(crd_v, [src3 + 1])
            z = plsc.load_gather(crd_v, [src3 + 2])
            valid = t < NUM_TYPES
            x = jnp.where(valid, x, zero_f)
            y = jnp.where(valid, y, zero_f)
            z = jnp.where(valid, z, zero_f)
            d3 = cb + pos * 3
            plsc.store_scatter(out_v, [d3], x)
            plsc.store_scatter(out_v, [d3 + 1], y)
            plsc.store_scatter(out_v, [d3 + 2], z)

    issue_in(0, 0)
    NPR = SPW // 2

    def pair_body(p, _):
        for k in (0, 1):
            ri = p * 2 + k

            @pl.when(ri + 1 < NPR)
            def _():
                issue_in(ri + 1, 1 - k)

            wait_in(ri, k)

            @pl.when(ri >= 2)
            def _():
                pltpu.make_async_copy(out_b[k], outc_hbm.at[rbase + ri - 2],
                                      sout[k]).wait()

            compute_sample(ri, k, 0)
            compute_sample(ri, k, 1)
            pltpu.async_copy(out_b[k], outc_hbm.at[rbase + ri], sout[k])
        return 0

    lax.fori_loop(0, NPR // 2, pair_body, 0)
    pltpu.make_async_copy(out_b[0], outc_hbm.at[rbase + NPR - 2], sout[0]).wait()
    pltpu.make_async_copy(out_b[1], outc_hbm.at[rbase + NPR - 1], sout[1]).wait()
    pltpu.sync_copy(co_v, cnts_hbm.at[pl.ds(base * NUM_TYPES, SPW * NUM_TYPES)])
    pltpu.sync_copy(of_v, offs_hbm.at[pl.ds(base * NUM_TYPES, SPW * NUM_TYPES)])


def kernel(input_coords_cpu, input_resnames, input_atomnames, num_atoms):
    types_packed = _pack_types_tc(
        input_resnames.astype(jnp.int32), input_atomnames.astype(jnp.int32)
    )
    out_coords, counts_flat, offsets_flat = _typed_coords_sc(
        input_coords_cpu.reshape(B // 2, 6 * M),
        types_packed.reshape(B // 2, 2 * Q),
        num_atoms.astype(jnp.int32),
    )
    return (
        out_coords.reshape(B, 3 * M),
        counts_flat.reshape(B, NUM_TYPES),
        offsets_flat.reshape(B, NUM_TYPES),
    )


# speedup vs baseline: 1.9463x; 1.9463x over previous
import functools

import jax
import jax.numpy as jnp
from jax import lax
from jax.experimental import pallas as pl
from jax.experimental.pallas import tpu as pltpu
from jax.experimental.pallas import tpu_sc as plsc

NUM_TYPES = 11
B = 1024
M = 4096
Q = M // 4
LANES = 16
NW = 32
SPW = B // NW
NVR = M // LANES
RB = 128

_mesh = plsc.VectorSubcoreMesh(core_axis_name="c", subcore_axis_name="s")


def _pack_body(res_ref, atm_ref, out_ref):
    s = res_ref[...] + atm_ref[...]
    s = s - jnp.where(s >= 44, 44, 0)
    s = s - jnp.where(s >= 22, 22, 0)
    s = s - jnp.where(s >= 11, 11, 0)
    out_ref[...] = (
        s[:, 0:Q]
        | (s[:, Q:2 * Q] << 8)
        | (s[:, 2 * Q:3 * Q] << 16)
        | (s[:, 3 * Q:4 * Q] << 24)
    )


_pack_types_tc = pl.pallas_call(
    _pack_body,
    grid=(B // RB,),
    in_specs=[
        pl.BlockSpec((RB, M), lambda i: (i, 0)),
        pl.BlockSpec((RB, M), lambda i: (i, 0)),
    ],
    out_specs=pl.BlockSpec((RB, Q), lambda i: (i, 0)),
    out_shape=jax.ShapeDtypeStruct((B, Q), jnp.int32),
)


@functools.partial(
    pl.kernel,
    out_type=[
        jax.ShapeDtypeStruct((B, 3 * M), jnp.float32),
        jax.ShapeDtypeStruct((B * NUM_TYPES,), jnp.int32),
        jax.ShapeDtypeStruct((B * NUM_TYPES,), jnp.int32),
    ],
    mesh=_mesh,
    scratch_types=[
        pltpu.VMEM((Q,), jnp.int32),
        pltpu.VMEM((Q,), jnp.int32),
        pltpu.VMEM((3 * M,), jnp.float32),
        pltpu.VMEM((3 * M,), jnp.float32),
        pltpu.VMEM((3 * M,), jnp.float32),
        pltpu.VMEM((3 * M,), jnp.float32),
        pltpu.VMEM((M,), jnp.int32),
        pltpu.VMEM((M,), jnp.int32),
        pltpu.VMEM((LANES,), jnp.int32),
        pltpu.VMEM((SPW,), jnp.int32),
        pltpu.VMEM((SPW * NUM_TYPES,), jnp.int32),
        pltpu.VMEM((SPW * NUM_TYPES,), jnp.int32),
        pltpu.SemaphoreType.DMA,
        pltpu.SemaphoreType.DMA,
        pltpu.SemaphoreType.DMA,
        pltpu.SemaphoreType.DMA,
    ],
    compiler_params=pltpu.CompilerParams(needs_layout_passes=False),
)
def _typed_coords_sc(
    crd_hbm, typ_hbm, na_hbm,
    outc_hbm, cnts_hbm, offs_hbm,
    typ_v0, typ_v1, crd_v0, crd_v1, out_v0, out_v1,
    enc_v, hst_v, off_v, na_v, co_v, of_v,
    sin0, sin1, sout0, sout1,
):
    typ_b = (typ_v0, typ_v1)
    crd_b = (crd_v0, crd_v1)
    out_b = (out_v0, out_v1)
    sin = (sin0, sin1)
    sout = (sout0, sout1)

    wid = lax.axis_index("c") * 16 + lax.axis_index("s")
    base = wid * SPW
    pltpu.sync_copy(na_hbm.at[pl.ds(base, SPW)], na_v)
    iota = lax.broadcasted_iota(jnp.int32, (LANES,), 0)
    zero_f = jnp.zeros((LANES,), jnp.float32)

    def issue_in(si, k):
        row = base + si
        pltpu.async_copy(typ_hbm.at[row], typ_b[k], sin[k])
        pltpu.async_copy(crd_hbm.at[row], crd_b[k], sin[k])

    def wait_in(si, k):
        row = base + si
        pltpu.make_async_copy(typ_hbm.at[row], typ_b[k], sin[k]).wait()
        pltpu.make_async_copy(crd_hbm.at[row], crd_b[k], sin[k]).wait()

    def compute_sample(si, k):
        typ_v, crd_v, out_v = typ_b[k], crd_b[k], out_b[k]
        na_sp = plsc.load_gather(na_v, [jnp.zeros((LANES,), jnp.int32) + si])

        @plsc.parallel_loop(0, NVR, unroll=8)
        def zero_hist(j):
            hst_v[pl.ds(j * LANES, LANES)] = jnp.zeros((LANES,), jnp.int32)

        @plsc.parallel_loop(0, NVR // 4)
        def pass_a(i):
            w = typ_v[pl.ds(i * LANES, LANES)]
            for q in range(4):
                j = q * (NVR // 4) + i
                t = (w >> (8 * q)) & 0xFF
                gi = j * LANES + iota
                t = jnp.where(gi < na_sp, t, NUM_TYPES)
                c1, lastm = plsc.scan_count(t)
                enc_v[pl.ds(j * LANES, LANES)] = t * LANES + (c1 - 1)
                plsc.store_scatter(hst_v, [j * LANES + t], c1, mask=lastm)

        def prefix(j, acc):
            sl = pl.ds(j * LANES, LANES)
            hv = hst_v[sl]
            hst_v[sl] = acc
            return acc + hv

        c = lax.fori_loop(0, NVR, prefix, jnp.zeros((LANES,), jnp.int32),
                          unroll=4)
        off = plsc.cumsum(c) - c
        off_v[...] = off
        dsti = si * NUM_TYPES + iota
        m11 = iota < NUM_TYPES
        plsc.store_scatter(co_v, [dsti], c, mask=m11)
        plsc.store_scatter(of_v, [dsti], off, mask=m11)

        @pl.when(si >= 2)
        def _():
            pltpu.make_async_copy(out_v, outc_hbm.at[base + si - 2],
                                  sout[k]).wait()

        @plsc.parallel_loop(0, NVR)
        def pass_b(j):
            sl = pl.ds(j * LANES, LANES)
            enc = enc_v[sl]
            t = enc >> 4
            pos = (plsc.load_gather(off_v, [t])
                   + plsc.load_gather(hst_v, [j * LANES + t])
                   + (enc & (LANES - 1)))
            src3 = (j * LANES) * 3 + iota * 3
            x = plsc.load_gather(crd_v, [src3])
            y = plsc.load_gather(crd_v, [src3 + 1])
            z = plsc.load_gather(crd_v, [src3 + 2])
            valid = t < NUM_TYPES
            x = jnp.where(valid, x, zero_f)
            y = jnp.where(valid, y, zero_f)
            z = jnp.where(valid, z, zero_f)
            d3 = pos * 3
            plsc.store_scatter(out_v, [d3], x)
            plsc.store_scatter(out_v, [d3 + 1], y)
            plsc.store_scatter(out_v, [d3 + 2], z)

        pltpu.async_copy(out_v, outc_hbm.at[base + si], sout[k])

    issue_in(0, 0)

    def pair_body(p, _):
        for k in (0, 1):
            si = p * 2 + k

            @pl.when(si + 1 < SPW)
            def _():
                issue_in(si + 1, 1 - k)

            wait_in(si, k)
            compute_sample(si, k)
        return 0

    lax.fori_loop(0, SPW // 2, pair_body, 0)
    pltpu.make_async_copy(out_b[0], outc_hbm.at[base + SPW - 2], sout[0]).wait()
    pltpu.make_async_copy(out_b[1], outc_hbm.at[base + SPW - 1], sout[1]).wait()
    pltpu.sync_copy(co_v, cnts_hbm.at[pl.ds(base * NUM_TYPES, SPW * NUM_TYPES)])
    pltpu.sync_copy(of_v, offs_hbm.at[pl.ds(base * NUM_TYPES, SPW * NUM_TYPES)])


def kernel(input_coords_cpu, input_resnames, input_atomnames, num_atoms):
    types_packed = _pack_types_tc(
        input_resnames.astype(jnp.int32), input_atomnames.astype(jnp.int32)
    )
    out_coords, counts_flat, offsets_flat = _typed_coords_sc(
        input_coords_cpu,
        types_packed,
        num_atoms.astype(jnp.int32),
    )
    return (
        out_coords,
        counts_flat.reshape(B, NUM_TYPES),
        offsets_flat.reshape(B, NUM_TYPES),
    )
